# Initial kernel scaffold; baseline (speedup 1.0000x reference)
import jax
import jax.numpy as jnp
from jax.experimental import pallas as pl


def kernel(fg_embed, pos_pair_index):
    raise NotImplementedError("write your pallas kernel here")


# speedup vs baseline: 3.1899x; 3.1899x over previous
import functools

import jax
import jax.numpy as jnp
from jax import lax
from jax.experimental import pallas as pl
from jax.experimental.pallas import tpu as pltpu
from jax.experimental.pallas import tpu_sc as plsc

V = 10000
D = 256
P = 160000

NC = 2
NS = 16
L = 16
NW = NC * NS
PPW = P // NW
CH = 200
NCH = PPW // CH

_mesh = plsc.VectorSubcoreMesh(core_axis_name="c", subcore_axis_name="s")


@functools.partial(
    pl.kernel,
    mesh=_mesh,
    out_type=jax.ShapeDtypeStruct((NW, L), jnp.float32),
    scratch_types=[
        pltpu.VMEM((CH,), jnp.int32),
        pltpu.VMEM((CH,), jnp.int32),
        pltpu.VMEM((CH, D), jnp.float32),
        pltpu.VMEM((CH, D), jnp.float32),
        pltpu.VMEM((L,), jnp.float32),
        pltpu.SemaphoreType.DMA,
        pltpu.SemaphoreType.DMA,
    ],
)
def _pair_sq_partials(table_hbm, idx1_hbm, idx2_hbm, out_hbm,
                      idx1_v, idx2_v, rows1_v, rows2_v, acc_v, sem1, sem2):
    wid = lax.axis_index("s") * NC + lax.axis_index("c")
    base = wid * PPW

    def chunk_body(t, total):
        off = base + t * CH
        pltpu.sync_copy(idx1_hbm.at[pl.ds(off, CH)], idx1_v)
        pltpu.sync_copy(idx2_hbm.at[pl.ds(off, CH)], idx2_v)
        g1 = pltpu.async_copy(table_hbm.at[idx1_v], rows1_v, sem1)
        g2 = pltpu.async_copy(table_hbm.at[idx2_v], rows2_v, sem2)
        g1.wait()
        g2.wait()

        def pair_body(p, acc):
            for c in range(D // L):
                a = rows1_v[p, pl.ds(c * L, L)]
                b = rows2_v[p, pl.ds(c * L, L)]
                dlt = a - b
                acc = acc + dlt * dlt
            return acc

        chunk_acc = lax.fori_loop(0, CH, pair_body,
                                  jnp.zeros((L,), jnp.float32))
        return total + chunk_acc

    total = lax.fori_loop(0, NCH, chunk_body, jnp.zeros((L,), jnp.float32))
    acc_v[...] = total
    pltpu.sync_copy(acc_v, out_hbm.at[wid])


def kernel(fg_embed, pos_pair_index):
    idx = pos_pair_index.astype(jnp.int32)
    parts = _pair_sq_partials(fg_embed, idx[0], idx[1])
    return jnp.sum(parts) / jnp.float32(P * D)


# speedup vs baseline: 4.0962x; 1.2841x over previous
import functools

import jax
import jax.numpy as jnp
from jax import lax
from jax.experimental import pallas as pl
from jax.experimental.pallas import tpu as pltpu
from jax.experimental.pallas import tpu_sc as plsc

V = 10000
D = 256
P = 160000

NC = 2
NS = 16
L = 16
NW = NC * NS
PPW = P // NW
CH = 200
NCH = PPW // CH
NACC = 16

_mesh = plsc.VectorSubcoreMesh(core_axis_name="c", subcore_axis_name="s")


@functools.partial(
    pl.kernel,
    mesh=_mesh,
    out_type=jax.ShapeDtypeStruct((NW, L), jnp.float32),
    scratch_types=[
        pltpu.VMEM((PPW,), jnp.int32),
        pltpu.VMEM((PPW,), jnp.int32),
        pltpu.VMEM((2, CH, D // 2), jnp.int32),
        pltpu.VMEM((2, CH, D // 2), jnp.int32),
        pltpu.VMEM((L,), jnp.float32),
        pltpu.SemaphoreType.DMA,
        pltpu.SemaphoreType.DMA,
    ],
)
def _pair_sq_partials(table_hbm, idx1_hbm, idx2_hbm, out_hbm,
                      idx1_v, idx2_v, rows1_v, rows2_v, acc_v, sem1, sem2):
    wid = lax.axis_index("s") * NC + lax.axis_index("c")
    base = wid * PPW
    pltpu.sync_copy(idx1_hbm.at[pl.ds(base, PPW)], idx1_v)
    pltpu.sync_copy(idx2_hbm.at[pl.ds(base, PPW)], idx2_v)

    def start(t, b):
        pltpu.async_copy(table_hbm.at[idx1_v.at[pl.ds(t * CH, CH)]],
                         rows1_v.at[b], sem1)
        pltpu.async_copy(table_hbm.at[idx2_v.at[pl.ds(t * CH, CH)]],
                         rows2_v.at[b], sem2)

    start(0, 0)

    def chunk_body(t, accs):
        b = lax.rem(t, 2)
        pltpu.make_async_copy(table_hbm.at[idx1_v.at[pl.ds(0, CH)]],
                              rows1_v.at[b], sem1).wait()
        pltpu.make_async_copy(table_hbm.at[idx2_v.at[pl.ds(0, CH)]],
                              rows2_v.at[b], sem2).wait()

        @pl.when(t + 1 < NCH)
        def _():
            start(t + 1, 1 - b)

        def pair_body(i, accs):
            accs = list(accs)
            for u in range(2):
                p = i * 2 + u
                for c in range(D // 32):
                    aw = rows1_v[b, p, pl.ds(c * L, L)]
                    bw = rows2_v[b, p, pl.ds(c * L, L)]
                    a_hi = lax.bitcast_convert_type(aw & jnp.int32(-65536), jnp.float32)
                    a_lo = lax.bitcast_convert_type(aw << jnp.int32(16), jnp.float32)
                    b_hi = lax.bitcast_convert_type(bw & jnp.int32(-65536), jnp.float32)
                    b_lo = lax.bitcast_convert_type(bw << jnp.int32(16), jnp.float32)
                    d_hi = a_hi - b_hi
                    d_lo = a_lo - b_lo
                    j = u * (D // 32) + c
                    accs[j] = accs[j] + d_hi * d_hi
                    accs[j] = accs[j] + d_lo * d_lo
            return tuple(accs)

        return lax.fori_loop(0, CH // 2, pair_body, accs)

    accs0 = tuple(jnp.zeros((L,), jnp.float32) for _ in range(NACC))
    accs = lax.fori_loop(0, NCH, chunk_body, accs0)

    total = accs[0]
    for j in range(1, NACC):
        total = total + accs[j]
    acc_v[...] = total
    pltpu.sync_copy(acc_v, out_hbm.at[wid])


def kernel(fg_embed, pos_pair_index):
    idx = pos_pair_index.astype(jnp.int32)
    table_bf = fg_embed.astype(jnp.bfloat16)
    table_w = jax.lax.bitcast_convert_type(
        table_bf.reshape(V, D // 2, 2), jnp.int32)
    parts = _pair_sq_partials(table_w, idx[0], idx[1])
    return jnp.sum(parts) / jnp.float32(P * D)


# speedup vs baseline: 7.3918x; 1.8046x over previous
import functools

import jax
import jax.numpy as jnp
from jax import lax
from jax.experimental import pallas as pl
from jax.experimental.pallas import tpu as pltpu
from jax.experimental.pallas import tpu_sc as plsc

V = 10000
D = 256
P = 160000

NC = 2
NS = 16
L = 16
NW = NC * NS
PPW = P // NW
CH = 200
NCH = PPW // CH
NACC = 16

_mesh = plsc.VectorSubcoreMesh(core_axis_name="c", subcore_axis_name="s")


@functools.partial(
    pl.kernel,
    mesh=_mesh,
    out_type=jax.ShapeDtypeStruct((NW, L), jnp.float32),
    scratch_types=[
        pltpu.VMEM((PPW,), jnp.int32),
        pltpu.VMEM((PPW,), jnp.int32),
        pltpu.VMEM((2, CH, D // 2), jnp.int32),
        pltpu.VMEM((2, CH, D // 2), jnp.int32),
        pltpu.VMEM((L,), jnp.float32),
        pltpu.SemaphoreType.DMA,
        pltpu.SemaphoreType.DMA,
    ],
)
def _pair_sq_partials(table_hbm, idx_hbm, out_hbm,
                      idx1_v, idx2_v, rows1_v, rows2_v, acc_v, sem1, sem2):
    wid = lax.axis_index("s") * NC + lax.axis_index("c")
    base = wid * PPW
    pltpu.sync_copy(idx_hbm.at[pl.ds(base, PPW)], idx1_v)
    pltpu.sync_copy(idx_hbm.at[pl.ds(P + base, PPW)], idx2_v)

    def start(t, b):
        pltpu.async_copy(table_hbm.at[idx1_v.at[pl.ds(t * CH, CH)]],
                         rows1_v.at[b], sem1)
        pltpu.async_copy(table_hbm.at[idx2_v.at[pl.ds(t * CH, CH)]],
                         rows2_v.at[b], sem2)

    start(0, 0)

    def chunk_body(t, accs):
        b = lax.rem(t, 2)
        pltpu.make_async_copy(table_hbm.at[idx1_v.at[pl.ds(0, CH)]],
                              rows1_v.at[b], sem1).wait()
        pltpu.make_async_copy(table_hbm.at[idx2_v.at[pl.ds(0, CH)]],
                              rows2_v.at[b], sem2).wait()

        @pl.when(t + 1 < NCH)
        def _():
            start(t + 1, 1 - b)

        def pair_body(i, accs):
            accs = list(accs)
            for u in range(2):
                p = i * 2 + u
                for c in range(D // 32):
                    aw = rows1_v[b, p, pl.ds(c * L, L)]
                    bw = rows2_v[b, p, pl.ds(c * L, L)]
                    a_hi = lax.bitcast_convert_type(aw & jnp.int32(-65536), jnp.float32)
                    a_lo = lax.bitcast_convert_type(aw << jnp.int32(16), jnp.float32)
                    b_hi = lax.bitcast_convert_type(bw & jnp.int32(-65536), jnp.float32)
                    b_lo = lax.bitcast_convert_type(bw << jnp.int32(16), jnp.float32)
                    d_hi = a_hi - b_hi
                    d_lo = a_lo - b_lo
                    j = u * (D // 32) + c
                    accs[j] = accs[j] + d_hi * d_hi
                    accs[j] = accs[j] + d_lo * d_lo
            return tuple(accs)

        return lax.fori_loop(0, CH // 2, pair_body, accs)

    accs0 = tuple(jnp.zeros((L,), jnp.float32) for _ in range(NACC))
    accs = lax.fori_loop(0, NCH, chunk_body, accs0)

    total = accs[0]
    for j in range(1, NACC):
        total = total + accs[j]
    acc_v[...] = total
    pltpu.sync_copy(acc_v, out_hbm.at[wid])


def kernel(fg_embed, pos_pair_index):
    idx = pos_pair_index.astype(jnp.int32)
    u = jax.lax.bitcast_convert_type(fg_embed, jnp.uint32)
    r = (u + jnp.uint32(0x7FFF) + ((u >> 16) & jnp.uint32(1))) >> 16
    w = r[:, : D // 2] | (r[:, D // 2:] << 16)
    table_w = jax.lax.bitcast_convert_type(w, jnp.int32)
    parts = _pair_sq_partials(table_w, idx.reshape(2 * P))
    return jnp.sum(parts) / jnp.float32(P * D)


# speedup vs baseline: 8.1714x; 1.1055x over previous
import functools

import jax
import jax.numpy as jnp
from jax import lax
from jax.experimental import pallas as pl
from jax.experimental.pallas import tpu as pltpu
from jax.experimental.pallas import tpu_sc as plsc

V = 10000
D = 256
P = 160000

NC = 2
NS = 16
L = 16
NW = NC * NS
PPW = P // NW
CH = 200
NCH = PPW // CH
NACC = 8

_mesh = plsc.VectorSubcoreMesh(core_axis_name="c", subcore_axis_name="s")


@functools.partial(
    pl.kernel,
    mesh=_mesh,
    out_type=jax.ShapeDtypeStruct((NW, L), jnp.float32),
    scratch_types=[
        pltpu.VMEM((PPW,), jnp.int32),
        pltpu.VMEM((PPW,), jnp.int32),
        pltpu.VMEM((2, CH, D // 2), jnp.int32),
        pltpu.VMEM((2, CH, D // 2), jnp.int32),
        pltpu.VMEM((L,), jnp.float32),
        pltpu.SemaphoreType.DMA,
        pltpu.SemaphoreType.DMA,
    ],
)
def _pair_sq_partials(table_hbm, idx_hbm, out_hbm,
                      idx1_v, idx2_v, rows1_v, rows2_v, acc_v, sem1, sem2):
    wid = lax.axis_index("s") * NC + lax.axis_index("c")
    base = wid * PPW
    pltpu.sync_copy(idx_hbm.at[pl.ds(base, PPW)], idx1_v)
    pltpu.sync_copy(idx_hbm.at[pl.ds(P + base, PPW)], idx2_v)

    def start(t, b):
        pltpu.async_copy(table_hbm.at[idx1_v.at[pl.ds(t * CH, CH)]],
                         rows1_v.at[b], sem1)
        pltpu.async_copy(table_hbm.at[idx2_v.at[pl.ds(t * CH, CH)]],
                         rows2_v.at[b], sem2)

    start(0, 0)

    def chunk_body(t, accs):
        b = lax.rem(t, 2)
        pltpu.make_async_copy(table_hbm.at[idx1_v.at[pl.ds(0, CH)]],
                              rows1_v.at[b], sem1).wait()
        pltpu.make_async_copy(table_hbm.at[idx2_v.at[pl.ds(0, CH)]],
                              rows2_v.at[b], sem2).wait()

        @pl.when(t + 1 < NCH)
        def _():
            start(t + 1, 1 - b)

        def pair_body(i, accs):
            accs = list(accs)
            for u in range(2):
                p = i * 2 + u
                for c in range(D // 32):
                    aw = rows1_v[b, p, pl.ds(c * L, L)]
                    bw = rows2_v[b, p, pl.ds(c * L, L)]
                    a_hi = lax.bitcast_convert_type(aw, jnp.float32)
                    a_lo = lax.bitcast_convert_type(aw << jnp.int32(16), jnp.float32)
                    b_hi = lax.bitcast_convert_type(bw, jnp.float32)
                    b_lo = lax.bitcast_convert_type(bw << jnp.int32(16), jnp.float32)
                    d_hi = a_hi - b_hi
                    d_lo = a_lo - b_lo
                    accs[c] = accs[c] + d_hi * d_hi
                    accs[c] = accs[c] + d_lo * d_lo
            return tuple(accs)

        return lax.fori_loop(0, CH // 2, pair_body, accs)

    accs0 = tuple(jnp.zeros((L,), jnp.float32) for _ in range(NACC))
    accs = lax.fori_loop(0, NCH, chunk_body, accs0)

    total = accs[0]
    for j in range(1, NACC):
        total = total + accs[j]
    acc_v[...] = total
    pltpu.sync_copy(acc_v, out_hbm.at[wid])


def kernel(fg_embed, pos_pair_index):
    idx = pos_pair_index.astype(jnp.int32)
    def rne(x):
        ub = jax.lax.bitcast_convert_type(x, jnp.uint32)
        return (ub + jnp.uint32(0x7FFF) + ((ub >> 16) & jnp.uint32(1))) >> 16

    r_lo = rne(fg_embed[:, : D // 2])
    r_hi = rne(fg_embed[:, D // 2:] * jnp.float32(1.0 - 2.0 ** -8))
    w = r_lo | (r_hi << 16)
    table_w = jax.lax.bitcast_convert_type(w, jnp.int32)
    parts = _pair_sq_partials(table_w, idx.reshape(2 * P))
    return jnp.sum(parts) / jnp.float32(P * D)


# speedup vs baseline: 8.2118x; 1.0050x over previous
import functools

import jax
import jax.numpy as jnp
from jax import lax
from jax.experimental import pallas as pl
from jax.experimental.pallas import tpu as pltpu
from jax.experimental.pallas import tpu_sc as plsc

V = 10000
D = 256
P = 160000

NC = 2
NS = 16
L = 16
NW = NC * NS
PPW = P // NW
CH = 200
NCH = PPW // CH
NACC = 4

_mesh = plsc.VectorSubcoreMesh(core_axis_name="c", subcore_axis_name="s")


@functools.partial(
    pl.kernel,
    mesh=_mesh,
    out_type=jax.ShapeDtypeStruct((NW, L), jnp.float32),
    scratch_types=[
        pltpu.VMEM((PPW,), jnp.int32),
        pltpu.VMEM((PPW,), jnp.int32),
        pltpu.VMEM((2, CH, D // 2), jnp.int32),
        pltpu.VMEM((2, CH, D // 2), jnp.int32),
        pltpu.VMEM((L,), jnp.float32),
        pltpu.SemaphoreType.DMA,
        pltpu.SemaphoreType.DMA,
    ],
)
def _pair_sq_partials(table_hbm, idx_hbm, out_hbm,
                      idx1_v, idx2_v, rows1_v, rows2_v, acc_v, sem1, sem2):
    wid = lax.axis_index("s") * NC + lax.axis_index("c")
    base = wid * PPW
    pltpu.sync_copy(idx_hbm.at[pl.ds(base, PPW)], idx1_v)
    pltpu.sync_copy(idx_hbm.at[pl.ds(P + base, PPW)], idx2_v)

    def start(t, b):
        pltpu.async_copy(table_hbm.at[idx1_v.at[pl.ds(t * CH, CH)]],
                         rows1_v.at[b], sem1)
        pltpu.async_copy(table_hbm.at[idx2_v.at[pl.ds(t * CH, CH)]],
                         rows2_v.at[b], sem2)

    start(0, 0)

    def chunk_body(t, accs):
        b = lax.rem(t, 2)
        pltpu.make_async_copy(table_hbm.at[idx1_v.at[pl.ds(0, CH)]],
                              rows1_v.at[b], sem1).wait()
        pltpu.make_async_copy(table_hbm.at[idx2_v.at[pl.ds(0, CH)]],
                              rows2_v.at[b], sem2).wait()

        @pl.when(t + 1 < NCH)
        def _():
            start(t + 1, 1 - b)

        rv1 = rows1_v.at[b]
        rv2 = rows2_v.at[b]

        def pair_body(i, accs):
            accs = list(accs)
            for u in range(2):
                p = i * 2 + u
                for c in range(D // 32):
                    aw = rv1[p, pl.ds(c * L, L)]
                    bw = rv2[p, pl.ds(c * L, L)]
                    a_hi = lax.bitcast_convert_type(aw, jnp.float32)
                    a_lo = lax.bitcast_convert_type(aw << jnp.int32(16), jnp.float32)
                    b_hi = lax.bitcast_convert_type(bw, jnp.float32)
                    b_lo = lax.bitcast_convert_type(bw << jnp.int32(16), jnp.float32)
                    d_hi = a_hi - b_hi
                    d_lo = a_lo - b_lo
                    accs[c % NACC] = accs[c % NACC] + d_hi * d_hi
                    accs[c % NACC] = accs[c % NACC] + d_lo * d_lo
            return tuple(accs)

        return lax.fori_loop(0, CH // 2, pair_body, accs)

    accs0 = tuple(jnp.zeros((L,), jnp.float32) for _ in range(NACC))
    accs = lax.fori_loop(0, NCH, chunk_body, accs0)

    total = accs[0]
    for j in range(1, NACC):
        total = total + accs[j]
    acc_v[...] = total
    pltpu.sync_copy(acc_v, out_hbm.at[wid])


def kernel(fg_embed, pos_pair_index):
    idx = pos_pair_index.astype(jnp.int32)
    def rne(x):
        ub = jax.lax.bitcast_convert_type(x, jnp.uint32)
        return (ub + jnp.uint32(0x7FFF) + ((ub >> 16) & jnp.uint32(1))) >> 16

    r_lo = rne(fg_embed[:, : D // 2])
    r_hi = rne(fg_embed[:, D // 2:] * jnp.float32(1.0 - 2.0 ** -8))
    w = r_lo | (r_hi << 16)
    table_w = jax.lax.bitcast_convert_type(w, jnp.int32)
    parts = _pair_sq_partials(table_w, idx.reshape(2 * P))
    return jnp.sum(parts) / jnp.float32(P * D)
